# baseline (speedup 1.0000x reference)
import functools

import jax
import jax.numpy as jnp
from jax import lax
from jax.experimental import pallas as pl
from jax.experimental.pallas import tpu as pltpu
from jax.experimental.pallas import tpu_sc as plsc

VOCAB = 100000
D = 128
S = 200
B = 4096

NC = 2
NS = 16
L = 16
BT = B // (NC * NS)
SC_B = B // NC
P = S // 2
NB = 2


def _mean_embed(sentence, table):
    mesh = plsc.VectorSubcoreMesh(core_axis_name="c", subcore_axis_name="s")

    @functools.partial(
        pl.kernel,
        mesh=mesh,
        out_type=jax.ShapeDtypeStruct((B, D), jnp.float32),
        scratch_types=[
            pltpu.VMEM((S, BT), jnp.int32),
            pltpu.VMEM((NB, BT, 2, D), jnp.float32),
            pltpu.VMEM((BT,), jnp.int32),
            pltpu.VMEM_SHARED((SC_B, 2, D), jnp.float32),
            [pltpu.SemaphoreType.DMA] * (2 * NB),
        ],
    )
    def k(sent_hbm, table_hbm, out_hbm, idx_v, rows_v, dst_v,
          accum_sh, gsems):
        cid = lax.axis_index("c")
        sid = lax.axis_index("s")
        tid = cid * NS + sid
        gbase = tid * BT
        lbase = sid * BT

        pltpu.sync_copy(sent_hbm.at[:, pl.ds(gbase, BT)], idx_v)

        for j in range(BT // L):
            dst_v[pl.ds(j * L, L)] = (
                jnp.full((L,), lbase + j * L, jnp.int32)
                + lax.iota(jnp.int32, L)
            )

        def gather_pair(p, b):
            for h in range(2):
                pltpu.async_copy(
                    table_hbm.at[idx_v.at[2 * p + h]],
                    rows_v.at[b].at[:, h], gsems[2 * b + h],
                )

        def wait_pair(b):
            for h in range(2):
                pltpu.make_async_copy(
                    table_hbm.at[idx_v.at[0]], rows_v.at[b].at[:, h],
                    gsems[2 * b + h],
                ).wait()

        for b in range(NB):
            gather_pair(b, b)

        wait_pair(0)
        pltpu.sync_copy(rows_v.at[0], accum_sh.at[dst_v])
        gather_pair(NB, 0)

        def body(g, carry):
            for b in range(NB):
                p = NB * g + b + 1
                bb = (b + 1) % NB

                @pl.when(p < P)
                def _step():
                    wait_pair(bb)
                    pltpu.sync_copy(rows_v.at[bb], accum_sh.at[dst_v],
                                    add=True)

                    @pl.when(p + NB < P)
                    def _refill():
                        gather_pair(p + NB, bb)
            return carry

        lax.fori_loop(0, (P - 1 + NB - 1) // NB, body, 0)

        acc_v = rows_v.at[0]
        pltpu.sync_copy(accum_sh.at[pl.ds(lbase, BT)], acc_v)
        inv = jnp.full((L,), 1.0 / S, jnp.float32)

        def sbody(r, carry):
            for j in range(D // L):
                acc_v[r, 0, pl.ds(j * L, L)] = (
                    acc_v[r, 0, pl.ds(j * L, L)]
                    + acc_v[r, 1, pl.ds(j * L, L)]
                ) * inv
            return carry

        lax.fori_loop(0, BT, sbody, 0)
        pltpu.sync_copy(acc_v.at[:, 0], out_hbm.at[pl.ds(gbase, BT)])

    return k(sentence, table)


def kernel(sentence, table):
    return _mean_embed(sentence, table)


# speedup vs baseline: 1.2303x; 1.2303x over previous
import functools

import jax
import jax.numpy as jnp
from jax import lax
from jax.experimental import pallas as pl
from jax.experimental.pallas import tpu as pltpu
from jax.experimental.pallas import tpu_sc as plsc

VOCAB = 100000
D = 128
S = 200
B = 4096

NC = 2
NS = 16
L = 16
BT = B // (NC * NS)
SC_B = B // NC


def _mean_embed(sentence, table):
    mesh = plsc.VectorSubcoreMesh(core_axis_name="c", subcore_axis_name="s")

    @functools.partial(
        pl.kernel,
        mesh=mesh,
        out_type=jax.ShapeDtypeStruct((B, D), jnp.float32),
        scratch_types=[
            pltpu.VMEM((S, BT), jnp.int32),
            pltpu.VMEM((4, BT, D), jnp.float32),
            pltpu.VMEM((BT,), jnp.int32),
            pltpu.VMEM_SHARED((SC_B, D), jnp.float32),
            [pltpu.SemaphoreType.DMA] * 4,
            pltpu.SemaphoreType.DMA,
        ],
    )
    def k(sent_hbm, table_hbm, out_hbm, idx_v, rows_v, dst_v,
          accum_sh, gsems, ssem):
        cid = lax.axis_index("c")
        sid = lax.axis_index("s")
        tid = cid * NS + sid
        gbase = tid * BT
        lbase = sid * BT

        pltpu.sync_copy(sent_hbm.at[:, pl.ds(gbase, BT)], idx_v)

        for j in range(BT // L):
            dst_v[pl.ds(j * L, L)] = (
                jnp.full((L,), lbase + j * L, jnp.int32)
                + lax.iota(jnp.int32, L)
            )

        NB = 4

        def wait_gather(b):
            pltpu.make_async_copy(
                table_hbm.at[idx_v.at[0]], rows_v.at[b], gsems[b]
            ).wait()

        def wait_scatter():
            pltpu.make_async_copy(
                rows_v.at[0], accum_sh.at[dst_v], ssem
            ).wait()

        for b in range(NB):
            pltpu.async_copy(table_hbm.at[idx_v.at[b]], rows_v.at[b], gsems[b])

        def body(g, carry):
            for b in range(NB):
                t = NB * g + b
                blag = (b - 1) % NB

                wait_gather(b)

                @pl.when(t >= 1)
                def _drain_refill():
                    wait_scatter()

                    @pl.when(t + NB - 1 < S)
                    def _refill():
                        pltpu.async_copy(
                            table_hbm.at[idx_v.at[t + NB - 1]],
                            rows_v.at[blag], gsems[blag],
                        )

                    pltpu.async_copy(
                        rows_v.at[b], accum_sh.at[dst_v], ssem, add=True
                    )

                @pl.when(t < 1)
                def _init():
                    pltpu.async_copy(rows_v.at[b], accum_sh.at[dst_v], ssem)
            return carry

        lax.fori_loop(0, S // NB, body, 0)
        wait_scatter()

        acc_v = rows_v.at[0]
        pltpu.sync_copy(accum_sh.at[pl.ds(lbase, BT)], acc_v)
        inv = jnp.full((L,), 1.0 / S, jnp.float32)

        def sbody(r, carry):
            for j in range(D // L):
                acc_v[r, pl.ds(j * L, L)] = acc_v[r, pl.ds(j * L, L)] * inv
            return carry

        lax.fori_loop(0, BT, sbody, 0)
        pltpu.sync_copy(acc_v, out_hbm.at[pl.ds(gbase, BT)])

    return k(sentence, table)


def kernel(sentence, table):
    return _mean_embed(sentence, table)


# speedup vs baseline: 1.2305x; 1.0002x over previous
import functools

import jax
import jax.numpy as jnp
from jax import lax
from jax.experimental import pallas as pl
from jax.experimental.pallas import tpu as pltpu
from jax.experimental.pallas import tpu_sc as plsc

VOCAB = 100000
D = 128
S = 200
B = 4096

NC = 2
NS = 16
L = 16
BT = B // (NC * NS)
SC_B = B // NC


def _mean_embed(sentence, table):
    mesh = plsc.VectorSubcoreMesh(core_axis_name="c", subcore_axis_name="s")

    @functools.partial(
        pl.kernel,
        mesh=mesh,
        out_type=jax.ShapeDtypeStruct((B, D), jnp.float32),
        scratch_types=[
            pltpu.VMEM((S, BT), jnp.int32),
            pltpu.VMEM((5, BT, D), jnp.float32),
            pltpu.VMEM((BT,), jnp.int32),
            pltpu.VMEM_SHARED((SC_B, D), jnp.float32),
            [pltpu.SemaphoreType.DMA] * 5,
            pltpu.SemaphoreType.DMA,
        ],
    )
    def k(sent_hbm, table_hbm, out_hbm, idx_v, rows_v, dst_v,
          accum_sh, gsems, ssem):
        cid = lax.axis_index("c")
        sid = lax.axis_index("s")
        tid = cid * NS + sid
        gbase = tid * BT
        lbase = sid * BT

        pltpu.sync_copy(sent_hbm.at[:, pl.ds(gbase, BT)], idx_v)

        for j in range(BT // L):
            dst_v[pl.ds(j * L, L)] = (
                jnp.full((L,), lbase + j * L, jnp.int32)
                + lax.iota(jnp.int32, L)
            )

        NB = 5

        def wait_gather(b):
            pltpu.make_async_copy(
                table_hbm.at[idx_v.at[0]], rows_v.at[b], gsems[b]
            ).wait()

        def wait_scatter():
            pltpu.make_async_copy(
                rows_v.at[0], accum_sh.at[dst_v], ssem
            ).wait()

        for b in range(NB):
            pltpu.async_copy(table_hbm.at[idx_v.at[b]], rows_v.at[b], gsems[b])

        def body(g, carry):
            for b in range(NB):
                t = NB * g + b
                blag = (b - 1) % NB

                @pl.when(t >= 1)
                def _drain_refill():
                    wait_scatter()

                    @pl.when(t + NB - 1 < S)
                    def _refill():
                        pltpu.async_copy(
                            table_hbm.at[idx_v.at[t + NB - 1]],
                            rows_v.at[blag], gsems[blag],
                        )

                wait_gather(b)

                @pl.when(t >= 1)
                def _add():
                    pltpu.async_copy(
                        rows_v.at[b], accum_sh.at[dst_v], ssem, add=True
                    )

                @pl.when(t < 1)
                def _init():
                    pltpu.async_copy(rows_v.at[b], accum_sh.at[dst_v], ssem)
            return carry

        lax.fori_loop(0, S // NB, body, 0)
        wait_scatter()

        acc_v = rows_v.at[0]
        pltpu.sync_copy(accum_sh.at[pl.ds(lbase, BT)], acc_v)
        inv = jnp.full((L,), 1.0 / S, jnp.float32)

        def sbody(r, carry):
            for j in range(D // L):
                acc_v[r, pl.ds(j * L, L)] = acc_v[r, pl.ds(j * L, L)] * inv
            return carry

        lax.fori_loop(0, BT, sbody, 0)
        pltpu.sync_copy(acc_v, out_hbm.at[pl.ds(gbase, BT)])

    return k(sentence, table)


def kernel(sentence, table):
    return _mean_embed(sentence, table)


# speedup vs baseline: 1.2330x; 1.0021x over previous
import functools

import jax
import jax.numpy as jnp
from jax import lax
from jax.experimental import pallas as pl
from jax.experimental.pallas import tpu as pltpu
from jax.experimental.pallas import tpu_sc as plsc

VOCAB = 100000
D = 128
S = 200
B = 4096

NC = 2
NS = 16
L = 16
BT = B // (NC * NS)
SC_B = B // NC


def _mean_embed(sentence, table):
    mesh = plsc.VectorSubcoreMesh(core_axis_name="c", subcore_axis_name="s")

    @functools.partial(
        pl.kernel,
        mesh=mesh,
        out_type=jax.ShapeDtypeStruct((B, D), jnp.float32),
        scratch_types=[
            pltpu.VMEM((S, BT), jnp.int32),
            pltpu.VMEM((5, BT, D), jnp.float32),
            pltpu.VMEM((BT,), jnp.int32),
            pltpu.VMEM_SHARED((SC_B, D), jnp.float32),
            [pltpu.SemaphoreType.DMA] * 5,
            pltpu.SemaphoreType.DMA,
            pltpu.SemaphoreType.DMA,
        ],
    )
    def k(sent_hbm, table_hbm, out_hbm, idx_v, rows_v, dst_v,
          accum_sh, gsems, ssem, stsem):
        cid = lax.axis_index("c")
        sid = lax.axis_index("s")
        tid = cid * NS + sid
        gbase = tid * BT
        lbase = sid * BT

        pltpu.async_copy(sent_hbm.at[pl.ds(8, S - 8), pl.ds(gbase, BT)],
                         idx_v.at[pl.ds(8, S - 8)], stsem)
        pltpu.sync_copy(sent_hbm.at[pl.ds(0, 8), pl.ds(gbase, BT)],
                        idx_v.at[pl.ds(0, 8)])

        for j in range(BT // L):
            dst_v[pl.ds(j * L, L)] = (
                jnp.full((L,), lbase + j * L, jnp.int32)
                + lax.iota(jnp.int32, L)
            )

        NB = 5

        def wait_gather(b):
            pltpu.make_async_copy(
                table_hbm.at[idx_v.at[0]], rows_v.at[b], gsems[b]
            ).wait()

        def wait_scatter():
            pltpu.make_async_copy(
                rows_v.at[0], accum_sh.at[dst_v], ssem
            ).wait()

        for b in range(NB):
            pltpu.async_copy(table_hbm.at[idx_v.at[b]], rows_v.at[b], gsems[b])

        def body(g, carry):
            for b in range(NB):
                t = NB * g + b
                blag = (b - 1) % NB

                @pl.when(t == 4)
                def _stwait():
                    pltpu.make_async_copy(
                        sent_hbm.at[pl.ds(8, S - 8), pl.ds(gbase, BT)],
                        idx_v.at[pl.ds(8, S - 8)], stsem,
                    ).wait()

                @pl.when(t >= 1)
                def _drain_refill():
                    wait_scatter()

                    @pl.when(t + NB - 1 < S)
                    def _refill():
                        pltpu.async_copy(
                            table_hbm.at[idx_v.at[t + NB - 1]],
                            rows_v.at[blag], gsems[blag],
                        )

                wait_gather(b)

                @pl.when(t >= 1)
                def _add():
                    pltpu.async_copy(
                        rows_v.at[b], accum_sh.at[dst_v], ssem, add=True
                    )

                @pl.when(t < 1)
                def _init():
                    pltpu.async_copy(rows_v.at[b], accum_sh.at[dst_v], ssem)
            return carry

        lax.fori_loop(0, S // NB, body, 0)
        wait_scatter()

        acc_v = rows_v.at[0]
        pltpu.sync_copy(accum_sh.at[pl.ds(lbase, BT)], acc_v)
        inv = jnp.full((L,), 1.0 / S, jnp.float32)

        def sbody(r, carry):
            for j in range(D // L):
                acc_v[r, pl.ds(j * L, L)] = acc_v[r, pl.ds(j * L, L)] * inv
            return carry

        lax.fori_loop(0, BT, sbody, 0)
        pltpu.sync_copy(acc_v, out_hbm.at[pl.ds(gbase, BT)])

    return k(sentence, table)


def kernel(sentence, table):
    return _mean_embed(sentence, table)
